# baseline (speedup 1.0000x reference)
import jax
import jax.numpy as jnp
from jax import lax
from jax.experimental import pallas as pl
from jax.experimental.pallas import tpu as pltpu
from jax.experimental.pallas import tpu_sc as plsc

N = 100000
VOCAB = 100
D = 16
OUT_D = 20

_info = plsc.get_sparse_core_info()
_NC, _NS, _L = _info.num_cores, _info.num_subcores, _info.num_lanes
_NW = _NC * _NS
PADV = 128
FULL = 3200
LAST = N - (_NW - 1) * FULL


def _sc_body(z_hbm, sd_hbm, df_hbm, cond_hbm, mult_hbm, tab_hbm, out_hbm,
             tab_v, idx_v, out_v, sem_in, sem_cols, sem_out):
    wid = lax.axis_index("s") * _NC + lax.axis_index("c")
    base = wid * FULL
    is_last = wid == _NW - 1

    def run(rows):
        in_cps = [
            pltpu.async_copy(tab_hbm, tab_v, sem_in),
            pltpu.async_copy(z_hbm.at[pl.ds(base, rows)],
                             idx_v.at[pl.ds(0, rows)], sem_in),
        ]
        col_cps = [
            pltpu.async_copy(col.at[pl.ds(base, rows)],
                             out_v.at[pl.ds((D + c) * FULL, rows)], sem_cols)
            for c, col in enumerate((sd_hbm, df_hbm, cond_hbm, mult_hbm))
        ]
        for cp in in_cps:
            cp.wait()

        half = rows // 2
        nblk_h = half // _L

        def gather_blocks(lo, hi):
            @plsc.parallel_loop(lo, hi, unroll=4)
            def _body(k):
                idx16 = idx_v[pl.ds(k * _L, _L)]
                vals = [plsc.load_gather(tab_v.at[pl.ds(g * PADV, PADV)],
                                         [idx16])
                        for g in range(D)]
                for g in range(D):
                    out_v[pl.ds(g * FULL + k * _L, _L)] = vals[g]

        gather_blocks(0, nblk_h)

        cps = [pltpu.async_copy(out_v.at[pl.ds(g * FULL, half)],
                                out_hbm.at[pl.ds(g * N + base, half)],
                                sem_out)
               for g in range(D)]

        gather_blocks(nblk_h, 2 * nblk_h)

        for cp in col_cps:
            cp.wait()
        cps += [pltpu.async_copy(out_v.at[pl.ds(g * FULL + half, half)],
                                 out_hbm.at[pl.ds(g * N + base + half, half)],
                                 sem_out)
                for g in range(D)]
        cps += [pltpu.async_copy(out_v.at[pl.ds(g * FULL, rows)],
                                 out_hbm.at[pl.ds(g * N + base, rows)],
                                 sem_out)
                for g in range(D, OUT_D)]
        for cp in cps:
            cp.wait()

    @pl.when(jnp.logical_not(is_last))
    def _():
        run(FULL)

    @pl.when(is_last)
    def _():
        run(LAST)


_sc_call = pl.kernel(
    _sc_body,
    mesh=plsc.VectorSubcoreMesh(core_axis_name="c", subcore_axis_name="s"),
    compiler_params=pltpu.CompilerParams(needs_layout_passes=False),
    out_type=jax.ShapeDtypeStruct((OUT_D * N,), jnp.float32),
    scratch_types=[
        pltpu.VMEM((D * PADV,), jnp.float32),
        pltpu.VMEM((FULL,), jnp.int32),
        pltpu.VMEM((OUT_D * FULL,), jnp.float32),
        pltpu.SemaphoreType.DMA,
        pltpu.SemaphoreType.DMA,
        pltpu.SemaphoreType.DMA,
    ],
)


def kernel(z, sd_coupling, d_filling_n, e_conductivity_n, d_filling_mult,
           z_embed_weight):
    tab_t = jnp.zeros((D, PADV), jnp.float32)
    tab_t = tab_t.at[:, :VOCAB].set(z_embed_weight.astype(jnp.float32).T)
    out = _sc_call(
        z.astype(jnp.int32),
        sd_coupling.reshape(N),
        d_filling_n.reshape(N),
        e_conductivity_n.reshape(N),
        d_filling_mult.reshape(N),
        tab_t.reshape(D * PADV),
    )
    return out.reshape(OUT_D, N).T


# speedup vs baseline: 1.0037x; 1.0037x over previous
import jax
import jax.numpy as jnp
from jax import lax
from jax.experimental import pallas as pl
from jax.experimental.pallas import tpu as pltpu
from jax.experimental.pallas import tpu_sc as plsc

N = 100000
VOCAB = 100
D = 16
OUT_D = 20

_info = plsc.get_sparse_core_info()
_NC, _NS, _L = _info.num_cores, _info.num_subcores, _info.num_lanes
_NW = _NC * _NS
PADV = 128
FULL = 3200
LAST = N - (_NW - 1) * FULL


def _sc_body(z_hbm, sd_hbm, df_hbm, cond_hbm, mult_hbm, tab_hbm, out_hbm,
             tab_v, idx_v, out_v, sem_in, sem_cols, sem_out):
    wid = lax.axis_index("s") * _NC + lax.axis_index("c")
    base = wid * FULL
    is_last = wid == _NW - 1

    def run(rows):
        in_cps = [
            pltpu.async_copy(tab_hbm, tab_v, sem_in),
            pltpu.async_copy(z_hbm.at[pl.ds(base, rows)],
                             idx_v.at[pl.ds(0, rows)], sem_in),
        ]
        col_cps = [
            pltpu.async_copy(col.at[pl.ds(base, rows)],
                             out_v.at[pl.ds((D + c) * FULL, rows)], sem_cols)
            for c, col in enumerate((sd_hbm, df_hbm, cond_hbm, mult_hbm))
        ]
        for cp in in_cps:
            cp.wait()

        def gather_blocks(lo, hi):
            @plsc.parallel_loop(lo, hi, unroll=2)
            def _body(k):
                idx16 = idx_v[pl.ds(k * _L, _L)]
                vals = [plsc.load_gather(tab_v.at[pl.ds(g * PADV, PADV)],
                                         [idx16])
                        for g in range(D)]
                for g in range(D):
                    out_v[pl.ds(g * FULL + k * _L, _L)] = vals[g]

        nsplit = 4 if rows % (4 * _L) == 0 else 2
        part = rows // nsplit
        nblk_p = part // _L
        cps = []
        for p in range(nsplit):
            gather_blocks(p * nblk_p, (p + 1) * nblk_p)
            cps += [pltpu.async_copy(
                out_v.at[pl.ds(g * FULL + p * part, part)],
                out_hbm.at[pl.ds(g * N + base + p * part, part)],
                sem_out)
                for g in range(D)]

        for cp in col_cps:
            cp.wait()
        cps += [pltpu.async_copy(out_v.at[pl.ds(g * FULL, rows)],
                                 out_hbm.at[pl.ds(g * N + base, rows)],
                                 sem_out)
                for g in range(D, OUT_D)]
        for cp in cps:
            cp.wait()

    @pl.when(jnp.logical_not(is_last))
    def _():
        run(FULL)

    @pl.when(is_last)
    def _():
        run(LAST)


_sc_call = pl.kernel(
    _sc_body,
    mesh=plsc.VectorSubcoreMesh(core_axis_name="c", subcore_axis_name="s"),
    compiler_params=pltpu.CompilerParams(needs_layout_passes=False),
    out_type=jax.ShapeDtypeStruct((OUT_D * N,), jnp.float32),
    scratch_types=[
        pltpu.VMEM((D * PADV,), jnp.float32),
        pltpu.VMEM((FULL,), jnp.int32),
        pltpu.VMEM((OUT_D * FULL,), jnp.float32),
        pltpu.SemaphoreType.DMA,
        pltpu.SemaphoreType.DMA,
        pltpu.SemaphoreType.DMA,
    ],
)


def kernel(z, sd_coupling, d_filling_n, e_conductivity_n, d_filling_mult,
           z_embed_weight):
    tab_t = jnp.zeros((D, PADV), jnp.float32)
    tab_t = tab_t.at[:, :VOCAB].set(z_embed_weight.astype(jnp.float32).T)
    out = _sc_call(
        z.astype(jnp.int32),
        sd_coupling.reshape(N),
        d_filling_n.reshape(N),
        e_conductivity_n.reshape(N),
        d_filling_mult.reshape(N),
        tab_t.reshape(D * PADV),
    )
    return out.reshape(OUT_D, N).T


# speedup vs baseline: 1.0090x; 1.0053x over previous
import jax
import jax.numpy as jnp
from jax import lax
from jax.experimental import pallas as pl
from jax.experimental.pallas import tpu as pltpu
from jax.experimental.pallas import tpu_sc as plsc

N = 100000
VOCAB = 100
D = 16
OUT_D = 20

_info = plsc.get_sparse_core_info()
_NC, _NS, _L = _info.num_cores, _info.num_subcores, _info.num_lanes
_NW = _NC * _NS
PADV = 128
FULL = 3200
LAST = N - (_NW - 1) * FULL


def _sc_body(z_hbm, sd_hbm, df_hbm, cond_hbm, mult_hbm, tab_hbm, out_hbm,
             tab_v, idx_v, out_v, sem_in, sem_cols, sem_out):
    wid = lax.axis_index("s") * _NC + lax.axis_index("c")
    base = wid * FULL
    is_last = wid == _NW - 1

    def run(rows):
        in_cps = [
            pltpu.async_copy(tab_hbm, tab_v, sem_in),
            pltpu.async_copy(z_hbm.at[pl.ds(base, rows)],
                             idx_v.at[pl.ds(0, rows)], sem_in),
        ]
        col_cps = [
            pltpu.async_copy(col.at[pl.ds(base, rows)],
                             out_v.at[pl.ds((D + c) * FULL, rows)], sem_cols)
            for c, col in enumerate((sd_hbm, df_hbm, cond_hbm, mult_hbm))
        ]
        for cp in in_cps:
            cp.wait()

        def gather_blocks(lo, hi):
            @plsc.parallel_loop(lo, hi, unroll=2)
            def _body(k):
                idx16 = idx_v[pl.ds(k * _L, _L)]
                vals = [plsc.load_gather(tab_v.at[pl.ds(g * PADV, PADV)],
                                         [idx16])
                        for g in range(D)]
                for g in range(D):
                    out_v[pl.ds(g * FULL + k * _L, _L)] = vals[g]

        nsplit = 2
        part = rows // nsplit
        nblk_p = part // _L
        cps = []
        for p in range(nsplit):
            gather_blocks(p * nblk_p, (p + 1) * nblk_p)
            cps += [pltpu.async_copy(
                out_v.at[pl.ds(g * FULL + p * part, part)],
                out_hbm.at[pl.ds(g * N + base + p * part, part)],
                sem_out)
                for g in range(D)]

        for cp in col_cps:
            cp.wait()
        cps += [pltpu.async_copy(out_v.at[pl.ds(g * FULL, rows)],
                                 out_hbm.at[pl.ds(g * N + base, rows)],
                                 sem_out)
                for g in range(D, OUT_D)]
        for cp in cps:
            cp.wait()

    @pl.when(jnp.logical_not(is_last))
    def _():
        run(FULL)

    @pl.when(is_last)
    def _():
        run(LAST)


_sc_call = pl.kernel(
    _sc_body,
    mesh=plsc.VectorSubcoreMesh(core_axis_name="c", subcore_axis_name="s"),
    compiler_params=pltpu.CompilerParams(needs_layout_passes=False),
    out_type=jax.ShapeDtypeStruct((OUT_D * N,), jnp.float32),
    scratch_types=[
        pltpu.VMEM((D * PADV,), jnp.float32),
        pltpu.VMEM((FULL,), jnp.int32),
        pltpu.VMEM((OUT_D * FULL,), jnp.float32),
        pltpu.SemaphoreType.DMA,
        pltpu.SemaphoreType.DMA,
        pltpu.SemaphoreType.DMA,
    ],
)


def kernel(z, sd_coupling, d_filling_n, e_conductivity_n, d_filling_mult,
           z_embed_weight):
    tab_t = jnp.zeros((D, PADV), jnp.float32)
    tab_t = tab_t.at[:, :VOCAB].set(z_embed_weight.astype(jnp.float32).T)
    out = _sc_call(
        z.astype(jnp.int32),
        sd_coupling.reshape(N),
        d_filling_n.reshape(N),
        e_conductivity_n.reshape(N),
        d_filling_mult.reshape(N),
        tab_t.reshape(D * PADV),
    )
    return out.reshape(OUT_D, N).T
